# baseline (speedup 1.0000x reference)
import jax
import jax.numpy as jnp
from jax import lax
from jax.experimental import pallas as pl
from jax.experimental.pallas import tpu as pltpu
from jax.experimental.pallas import tpu_sc as plsc

N = 10000
DC = 64
E = 320000
NC = 2
NS = 16
CHUNK = 128
NBUF = 4
PF = 2
DB = 2
CPW = 160
KOUT = CPW // NBUF
EP = NS * CPW * CHUNK
NPAD = 10240
ZROWS = NPAD // NS
OSTEP = 624
OROWS = 640
DEGW = 16


_MESH = plsc.VectorSubcoreMesh(
    core_axis_name="c", subcore_axis_name="s", num_cores=NC, num_subcores=NS)

_SC_PARAMS = pltpu.CompilerParams(use_tc_tiling_on_sc=False)


def _make_sc_pass(with_deg):
    out_types = [jax.ShapeDtypeStruct((N, DC), jnp.float32)]
    scratch = [pltpu.VMEM((CPW, CHUNK), jnp.int32),
               pltpu.VMEM((CPW, CHUNK), jnp.int32)]
    scratch += [pltpu.VMEM((CHUNK, DC), jnp.float32) for _ in range(NBUF)]
    scratch.append(pltpu.VMEM_SHARED((NPAD, DC), jnp.float32))
    scratch += [pltpu.SemaphoreType.DMA for _ in range(2 * NBUF)]
    if with_deg:
        out_types.append(jax.ShapeDtypeStruct((N, DEGW), jnp.float32))
        scratch += [pltpu.VMEM((CHUNK, DEGW), jnp.float32),
                    pltpu.VMEM_SHARED((NPAD, DEGW), jnp.float32)]
        scratch += [pltpu.SemaphoreType.DMA for _ in range(DB)]

    def body(*args):
        if with_deg:
            (table, srcs, dsts, z64, z16, ones, agg_out, deg_out,
             src_v, dst_v, *rest) = args
        else:
            (table, srcs, dsts, z64, agg_out, src_v, dst_v, *rest) = args
        rows = rest[0:NBUF]
        agg_sh = rest[NBUF]
        g = rest[NBUF + 1:2 * NBUF + 1]
        s = rest[2 * NBUF + 1:3 * NBUF + 1]
        if with_deg:
            ones_v = rest[3 * NBUF + 1]
            deg_sh = rest[3 * NBUF + 2]
            d = rest[3 * NBUF + 3:3 * NBUF + 3 + DB]
        c = lax.axis_index("c")
        sid = lax.axis_index("s")

        @pl.when(c == 0)
        def _work():
            stage = [
                pltpu.make_async_copy(
                    z64.at[pl.ds(sid * ZROWS, ZROWS)],
                    agg_sh.at[pl.ds(sid * ZROWS, ZROWS)], g[0]),
                pltpu.make_async_copy(srcs.at[sid], src_v, g[1]),
                pltpu.make_async_copy(dsts.at[sid], dst_v, g[2]),
            ]
            if with_deg:
                stage += [
                    pltpu.make_async_copy(
                        z16.at[pl.ds(sid * ZROWS, ZROWS)],
                        deg_sh.at[pl.ds(sid * ZROWS, ZROWS)], s[0]),
                    pltpu.make_async_copy(ones, ones_v, s[1]),
                ]
            for cp in stage:
                cp.start()
            for cp in stage:
                cp.wait()
            plsc.subcore_barrier()

            def gissue(row, b):
                pltpu.async_copy(table.at[src_v.at[row]], rows[b], g[b])

            def gwait(row, b):
                pltpu.make_async_copy(table.at[src_v.at[row]], rows[b],
                                      g[b]).wait()

            def sissue(row, b):
                pltpu.async_copy(rows[b], agg_sh.at[dst_v.at[row]], s[b],
                                 add=True)

            def swait(row, b):
                pltpu.make_async_copy(rows[b], agg_sh.at[dst_v.at[row]],
                                      s[b]).wait()

            def dissue(row, b):
                pltpu.async_copy(ones_v, deg_sh.at[dst_v.at[row]], d[b],
                                 add=True)

            def dwait(row, b):
                pltpu.make_async_copy(ones_v, deg_sh.at[dst_v.at[row]],
                                      d[b]).wait()

            def step(j, b, prefetch=True, dpipe=with_deg, dskip_wait=False):
                gwait(j, b)
                sissue(j, b)
                if dpipe:
                    if not dskip_wait:
                        dwait(j - DB, b % DB)
                    dissue(j, b % DB)
                if prefetch:
                    b2 = (b + PF) % NBUF
                    swait(j - PF, b2)
                    gissue(j + PF, b2)

            for b in range(PF):
                gissue(b, b)
            for b in range(NBUF):
                step(b, b, prefetch=b >= PF, dskip_wait=b < DB)
                if b < PF:
                    gissue(b + PF, (b + PF) % NBUF)

            def outer(k, carry):
                base = k * NBUF
                for b in range(NBUF):
                    step(base + b, b)
                return carry

            lax.fori_loop(1, KOUT - 1, outer, 0)

            last = (KOUT - 1) * NBUF
            for b in range(NBUF):
                step(last + b, b, prefetch=b < PF)
            for b in range(NBUF):
                swait(last + b, b)
            if with_deg:
                for b in range(DB):
                    dwait(last + NBUF - DB + b, b)

            plsc.subcore_barrier()
            out_cps = [
                pltpu.make_async_copy(
                    agg_sh.at[pl.ds(sid * OSTEP, OROWS)],
                    agg_out.at[pl.ds(sid * OSTEP, OROWS)], g[0]),
            ]
            if with_deg:
                out_cps.append(pltpu.make_async_copy(
                    deg_sh.at[pl.ds(sid * OSTEP, OROWS)],
                    deg_out.at[pl.ds(sid * OSTEP, OROWS)], g[1]))
            for cp in out_cps:
                cp.start()
            for cp in out_cps:
                cp.wait()

    return pl.kernel(
        body,
        out_type=tuple(out_types) if with_deg else out_types[0],
        mesh=_MESH,
        compiler_params=_SC_PARAMS,
        scratch_types=scratch,
    )


_sc_agg_deg = _make_sc_pass(True)
_sc_agg = _make_sc_pass(False)


def _tc_combine_body(agg_ref, deg_ref, xs_ref, w_ref, b_ref, y_ref):
    agg = agg_ref[...]
    deg = jnp.sum(deg_ref[...], axis=1, keepdims=True) * (1.0 / DEGW)
    r = 1.0 / jnp.maximum(deg, 1.0)
    h = jnp.dot(agg * r, w_ref[...], preferred_element_type=jnp.float32)
    y_ref[...] = xs_ref[...] + jnp.maximum(h + b_ref[...], 0.0)


_tc_combine = pl.pallas_call(
    _tc_combine_body,
    out_shape=jax.ShapeDtypeStruct((N, DC), jnp.float32),
)


def _partition(idx, fill):
    pad = EP - E
    flat = jnp.concatenate([idx, jnp.full((pad,), fill, jnp.int32)])
    return flat.reshape(NS, CPW, CHUNK)


def kernel(x, edge_index, W0, b0, W1, b1):
    xs0 = x[:, :DC]
    xs1 = x[:, DC:]
    srcp = _partition(edge_index[0], 0)
    dstp = _partition(edge_index[1], N)
    z64 = jnp.zeros((NPAD, DC), jnp.float32)
    z16 = jnp.zeros((NPAD, DEGW), jnp.float32)
    ones = jnp.ones((CHUNK, DEGW), jnp.float32)
    agg0, degp = _sc_agg_deg(xs1, srcp, dstp, z64, z16, ones)
    y0 = _tc_combine(agg0, degp, xs0, W0, b0.reshape(1, DC))
    agg1 = _sc_agg(y0, srcp, dstp, z64)
    y1 = _tc_combine(agg1, degp, xs1, W1, b1.reshape(1, DC))
    return jnp.concatenate([y0, y1], axis=-1)


# speedup vs baseline: 1.2335x; 1.2335x over previous
import jax
import jax.numpy as jnp
from jax import lax
from jax.experimental import pallas as pl
from jax.experimental.pallas import tpu as pltpu
from jax.experimental.pallas import tpu_sc as plsc

N = 10000
DC = 64
E = 320000
NC = 2
NS = 16
NW = NC * NS
CHUNK = 128
NBUF = 4
PF = 2
DB = 2
CA = 124
CB = 36
KA = CA // NBUF
KB = CB // NBUF
CMAX = max(CA, CB)
EP = NS * (CA + CB) * CHUNK
NPAD = 10240
ZROWS = NPAD // NS
OSTEP = 624
OROWS = 640
DEGW = 16


_MESH = plsc.VectorSubcoreMesh(
    core_axis_name="c", subcore_axis_name="s", num_cores=NC, num_subcores=NS)

_SC_PARAMS = pltpu.CompilerParams(use_tc_tiling_on_sc=False)


def _make_sc_pass(with_deg):
    out_types = [jax.ShapeDtypeStruct((NC * N, DC), jnp.float32)]
    scratch = [pltpu.VMEM((CMAX, CHUNK), jnp.int32),
               pltpu.VMEM((CMAX, CHUNK), jnp.int32)]
    scratch += [pltpu.VMEM((CHUNK, DC), jnp.float32) for _ in range(NBUF)]
    scratch.append(pltpu.VMEM_SHARED((NPAD, DC), jnp.float32))
    scratch += [pltpu.SemaphoreType.DMA for _ in range(2 * NBUF)]
    if with_deg:
        out_types.append(jax.ShapeDtypeStruct((NC * N, DEGW), jnp.float32))
        scratch += [pltpu.VMEM((CHUNK, DEGW), jnp.float32),
                    pltpu.VMEM((CHUNK, DEGW), jnp.float32),
                    pltpu.VMEM_SHARED((NPAD, DEGW), jnp.float32)]
        scratch += [pltpu.SemaphoreType.DMA for _ in range(DB)]

    def body(*args):
        if with_deg:
            (table, srcs, dsts, agg_out, deg_out,
             src_v, dst_v, *rest) = args
        else:
            (table, srcs, dsts, agg_out, src_v, dst_v, *rest) = args
        rows = rest[0:NBUF]
        agg_sh = rest[NBUF]
        g = rest[NBUF + 1:2 * NBUF + 1]
        s = rest[2 * NBUF + 1:3 * NBUF + 1]
        if with_deg:
            ones_v = rest[3 * NBUF + 1]
            zb16 = rest[3 * NBUF + 2]
            deg_sh = rest[3 * NBUF + 3]
            d = rest[3 * NBUF + 4:3 * NBUF + 4 + DB]
        c = lax.axis_index("c")
        sid = lax.axis_index("s")
        wid = sid * NC + c
        kend = jnp.where(c == 0, KA, KB)

        def _idx_cps(nch):
            return [
                pltpu.make_async_copy(srcs.at[pl.ds(wid * CMAX, nch)],
                                      src_v.at[pl.ds(0, nch)], g[1]),
                pltpu.make_async_copy(dsts.at[pl.ds(wid * CMAX, nch)],
                                      dst_v.at[pl.ds(0, nch)], g[2]),
            ]

        @pl.when(c == 0)
        def _issue_a():
            for cp in _idx_cps(CA):
                cp.start()

        @pl.when(c != 0)
        def _issue_b():
            for cp in _idx_cps(CB):
                cp.start()

        zv = jnp.zeros((16,), jnp.float32)

        def _zfill(i, carry):
            for q in range(DC // 16):
                rows[0][i, pl.ds(q * 16, 16)] = zv
            if with_deg:
                zb16[i, :] = zv
                ones_v[i, :] = zv + 1.0
            return carry

        lax.fori_loop(0, CHUNK, _zfill, 0)
        for q in range(ZROWS // CHUNK):
            pltpu.sync_copy(
                rows[0], agg_sh.at[pl.ds(sid * ZROWS + q * CHUNK, CHUNK)])
            if with_deg:
                pltpu.sync_copy(
                    zb16, deg_sh.at[pl.ds(sid * ZROWS + q * CHUNK, CHUNK)])

        @pl.when(c == 0)
        def _wait_a():
            for cp in _idx_cps(CA):
                cp.wait()

        @pl.when(c != 0)
        def _wait_b():
            for cp in _idx_cps(CB):
                cp.wait()

        plsc.subcore_barrier()

        def gissue(row, b):
            pltpu.async_copy(table.at[src_v.at[row]], rows[b], g[b])

        def gwait(row, b):
            pltpu.make_async_copy(table.at[src_v.at[row]], rows[b],
                                  g[b]).wait()

        def sissue(row, b):
            pltpu.async_copy(rows[b], agg_sh.at[dst_v.at[row]], s[b],
                             add=True)

        def swait(row, b):
            pltpu.make_async_copy(rows[b], agg_sh.at[dst_v.at[row]],
                                  s[b]).wait()

        def dissue(row, b):
            pltpu.async_copy(ones_v, deg_sh.at[dst_v.at[row]], d[b],
                             add=True)

        def dwait(row, b):
            pltpu.make_async_copy(ones_v, deg_sh.at[dst_v.at[row]],
                                  d[b]).wait()

        def step(j, b, prefetch=True, dpipe=with_deg, dskip_wait=False):
            gwait(j, b)
            sissue(j, b)
            if dpipe:
                if not dskip_wait:
                    dwait(j - DB, b % DB)
                dissue(j, b % DB)
            if prefetch:
                b2 = (b + PF) % NBUF
                swait(j - PF, b2)
                gissue(j + PF, b2)

        for b in range(PF):
            gissue(b, b)
        for b in range(NBUF):
            step(b, b, prefetch=b >= PF, dskip_wait=b < DB)
            if b < PF:
                gissue(b + PF, (b + PF) % NBUF)

        def outer(k, carry):
            base = k * NBUF
            for b in range(NBUF):
                step(base + b, b)
            return carry

        lax.fori_loop(1, kend - 1, outer, 0)

        last = (kend - 1) * NBUF
        for b in range(NBUF):
            step(last + b, b, prefetch=b < PF)
        for b in range(NBUF):
            swait(last + b, b)
        if with_deg:
            for b in range(DB):
                dwait(last + NBUF - DB + b, b)

        plsc.subcore_barrier()
        out_cps = [
            pltpu.make_async_copy(
                agg_sh.at[pl.ds(sid * OSTEP, OROWS)],
                agg_out.at[pl.ds(c * N + sid * OSTEP, OROWS)], g[0]),
        ]
        if with_deg:
            out_cps.append(pltpu.make_async_copy(
                deg_sh.at[pl.ds(sid * OSTEP, OROWS)],
                deg_out.at[pl.ds(c * N + sid * OSTEP, OROWS)], g[1]))
        for cp in out_cps:
            cp.start()
        for cp in out_cps:
            cp.wait()

    return pl.kernel(
        body,
        out_type=tuple(out_types) if with_deg else out_types[0],
        mesh=_MESH,
        compiler_params=_SC_PARAMS,
        scratch_types=scratch,
    )


_sc_agg_deg = _make_sc_pass(True)
_sc_agg = _make_sc_pass(False)


def _tc_combine_body(aggp_ref, degp_ref, xs_ref, w_ref, b_ref, y_ref):
    agg = aggp_ref[pl.ds(0, N), :] + aggp_ref[pl.ds(N, N), :]
    d = degp_ref[pl.ds(0, N), :] + degp_ref[pl.ds(N, N), :]
    deg = jnp.sum(d, axis=1, keepdims=True) * (1.0 / DEGW)
    r = 1.0 / jnp.maximum(deg, 1.0)
    h = jnp.dot(agg * r, w_ref[...], preferred_element_type=jnp.float32)
    y_ref[...] = xs_ref[...] + jnp.maximum(h + b_ref[...], 0.0)


_tc_combine = pl.pallas_call(
    _tc_combine_body,
    out_shape=jax.ShapeDtypeStruct((N, DC), jnp.float32),
)


def _partition(idx, fill):
    pad = EP - E
    flat = jnp.concatenate([idx, jnp.full((pad,), fill, jnp.int32)])
    e0 = flat[:NS * CA * CHUNK].reshape(NS, CA, CHUNK)
    e1 = flat[NS * CA * CHUNK:].reshape(NS, CB, CHUNK)
    if CB < CMAX:
        e1 = jnp.concatenate(
            [e1, jnp.zeros((NS, CMAX - CB, CHUNK), jnp.int32)], axis=1)
    if CA < CMAX:
        e0 = jnp.concatenate(
            [e0, jnp.zeros((NS, CMAX - CA, CHUNK), jnp.int32)], axis=1)
    return jnp.stack([e0, e1], axis=1).reshape(NW * CMAX, CHUNK)


def kernel(x, edge_index, W0, b0, W1, b1):
    xs0 = x[:, :DC]
    xs1 = x[:, DC:]
    srcp = _partition(edge_index[0], 0)
    dstp = _partition(edge_index[1], N)
    aggp0, degp = _sc_agg_deg(xs1, srcp, dstp)
    y0 = _tc_combine(aggp0, degp, xs0, W0, b0.reshape(1, DC))
    aggp1 = _sc_agg(y0, srcp, dstp)
    y1 = _tc_combine(aggp1, degp, xs1, W1, b1.reshape(1, DC))
    return jnp.concatenate([y0, y1], axis=-1)


# speedup vs baseline: 1.2860x; 1.0425x over previous
import jax
import jax.numpy as jnp
from jax import lax
from jax.experimental import pallas as pl
from jax.experimental.pallas import tpu as pltpu
from jax.experimental.pallas import tpu_sc as plsc

N = 10000
DC = 64
E = 320000
NC = 2
NS = 16
NW = NC * NS
CHUNK = 128
NBUF = 4
PF = 2
DB = 2
CA = 144
CB = 16
KA = CA // NBUF
KB = CB // NBUF
CMAX = max(CA, CB)
EP = NS * (CA + CB) * CHUNK
NPAD = 10240
ZROWS = NPAD // NS
OSTEP = 624
OROWS = 640
DEGW = 16


_MESH = plsc.VectorSubcoreMesh(
    core_axis_name="c", subcore_axis_name="s", num_cores=NC, num_subcores=NS)

_SC_PARAMS = pltpu.CompilerParams(use_tc_tiling_on_sc=False)


def _make_sc_pass(with_deg):
    out_types = [jax.ShapeDtypeStruct((NC * N, DC), jnp.float32)]
    scratch = [pltpu.VMEM((CMAX, CHUNK), jnp.int32),
               pltpu.VMEM((CMAX, CHUNK), jnp.int32)]
    scratch += [pltpu.VMEM((CHUNK, DC), jnp.float32) for _ in range(NBUF)]
    scratch.append(pltpu.VMEM_SHARED((NPAD, DC), jnp.float32))
    scratch += [pltpu.SemaphoreType.DMA for _ in range(2 * NBUF)]
    if with_deg:
        out_types.append(jax.ShapeDtypeStruct((NC * N, DEGW), jnp.float32))
        scratch += [pltpu.VMEM((CHUNK, DEGW), jnp.float32),
                    pltpu.VMEM((CHUNK, DEGW), jnp.float32),
                    pltpu.VMEM_SHARED((NPAD, DEGW), jnp.float32)]
        scratch += [pltpu.SemaphoreType.DMA for _ in range(DB)]

    def body(*args):
        if with_deg:
            (table, srcs, dsts, agg_out, deg_out,
             src_v, dst_v, *rest) = args
        else:
            (table, srcs, dsts, agg_out, src_v, dst_v, *rest) = args
        rows = rest[0:NBUF]
        agg_sh = rest[NBUF]
        g = rest[NBUF + 1:2 * NBUF + 1]
        s = rest[2 * NBUF + 1:3 * NBUF + 1]
        if with_deg:
            ones_v = rest[3 * NBUF + 1]
            zb16 = rest[3 * NBUF + 2]
            deg_sh = rest[3 * NBUF + 3]
            d = rest[3 * NBUF + 4:3 * NBUF + 4 + DB]
        c = lax.axis_index("c")
        sid = lax.axis_index("s")
        wid = sid * NC + c
        kend = jnp.where(c == 0, KA, KB)

        def _idx_cps(nch):
            return [
                pltpu.make_async_copy(srcs.at[pl.ds(wid * CMAX, nch)],
                                      src_v.at[pl.ds(0, nch)], g[1]),
                pltpu.make_async_copy(dsts.at[pl.ds(wid * CMAX, nch)],
                                      dst_v.at[pl.ds(0, nch)], g[2]),
            ]

        @pl.when(c == 0)
        def _issue_a():
            for cp in _idx_cps(CA):
                cp.start()

        @pl.when(c != 0)
        def _issue_b():
            for cp in _idx_cps(CB):
                cp.start()

        zv = jnp.zeros((16,), jnp.float32)

        def _zfill(i, carry):
            for q in range(DC // 16):
                rows[0][i, pl.ds(q * 16, 16)] = zv
            if with_deg:
                zb16[i, :] = zv
                ones_v[i, :] = zv + 1.0
            return carry

        lax.fori_loop(0, CHUNK, _zfill, 0)
        for q in range(ZROWS // CHUNK):
            pltpu.sync_copy(
                rows[0], agg_sh.at[pl.ds(sid * ZROWS + q * CHUNK, CHUNK)])
            if with_deg:
                pltpu.sync_copy(
                    zb16, deg_sh.at[pl.ds(sid * ZROWS + q * CHUNK, CHUNK)])

        @pl.when(c == 0)
        def _wait_a():
            for cp in _idx_cps(CA):
                cp.wait()

        @pl.when(c != 0)
        def _wait_b():
            for cp in _idx_cps(CB):
                cp.wait()

        plsc.subcore_barrier()

        def gissue(row, b):
            pltpu.async_copy(table.at[src_v.at[row]], rows[b], g[b])

        def gwait(row, b):
            pltpu.make_async_copy(table.at[src_v.at[row]], rows[b],
                                  g[b]).wait()

        def sissue(row, b):
            pltpu.async_copy(rows[b], agg_sh.at[dst_v.at[row]], s[b],
                             add=True)

        def swait(row, b):
            pltpu.make_async_copy(rows[b], agg_sh.at[dst_v.at[row]],
                                  s[b]).wait()

        def dissue(row, b):
            pltpu.async_copy(ones_v, deg_sh.at[dst_v.at[row]], d[b],
                             add=True)

        def dwait(row, b):
            pltpu.make_async_copy(ones_v, deg_sh.at[dst_v.at[row]],
                                  d[b]).wait()

        def step(j, b, prefetch=True, dpipe=with_deg, dskip_wait=False):
            gwait(j, b)
            sissue(j, b)
            if dpipe:
                if not dskip_wait:
                    dwait(j - DB, b % DB)
                dissue(j, b % DB)
            if prefetch:
                b2 = (b + PF) % NBUF
                swait(j - PF, b2)
                gissue(j + PF, b2)

        for b in range(PF):
            gissue(b, b)
        for b in range(NBUF):
            step(b, b, prefetch=b >= PF, dskip_wait=b < DB)
            if b < PF:
                gissue(b + PF, (b + PF) % NBUF)

        def outer(k, carry):
            base = k * NBUF
            for b in range(NBUF):
                step(base + b, b)
            return carry

        lax.fori_loop(1, kend - 1, outer, 0)

        last = (kend - 1) * NBUF
        for b in range(NBUF):
            step(last + b, b, prefetch=b < PF)
        for b in range(NBUF):
            swait(last + b, b)
        if with_deg:
            for b in range(DB):
                dwait(last + NBUF - DB + b, b)

        plsc.subcore_barrier()
        out_cps = [
            pltpu.make_async_copy(
                agg_sh.at[pl.ds(sid * OSTEP, OROWS)],
                agg_out.at[pl.ds(c * N + sid * OSTEP, OROWS)], g[0]),
        ]
        if with_deg:
            out_cps.append(pltpu.make_async_copy(
                deg_sh.at[pl.ds(sid * OSTEP, OROWS)],
                deg_out.at[pl.ds(c * N + sid * OSTEP, OROWS)], g[1]))
        for cp in out_cps:
            cp.start()
        for cp in out_cps:
            cp.wait()

    return pl.kernel(
        body,
        out_type=tuple(out_types) if with_deg else out_types[0],
        mesh=_MESH,
        compiler_params=_SC_PARAMS,
        scratch_types=scratch,
    )


_sc_agg_deg = _make_sc_pass(True)
_sc_agg = _make_sc_pass(False)


def _tc_combine_body(aggp_ref, degp_ref, xs_ref, w_ref, b_ref, y_ref):
    agg = aggp_ref[pl.ds(0, N), :] + aggp_ref[pl.ds(N, N), :]
    d = degp_ref[pl.ds(0, N), :] + degp_ref[pl.ds(N, N), :]
    deg = jnp.sum(d, axis=1, keepdims=True) * (1.0 / DEGW)
    r = 1.0 / jnp.maximum(deg, 1.0)
    h = jnp.dot(agg * r, w_ref[...], preferred_element_type=jnp.float32)
    y_ref[...] = xs_ref[...] + jnp.maximum(h + b_ref[...], 0.0)


_tc_combine = pl.pallas_call(
    _tc_combine_body,
    out_shape=jax.ShapeDtypeStruct((N, DC), jnp.float32),
)


def _partition(idx, fill):
    pad = EP - E
    flat = jnp.concatenate([idx, jnp.full((pad,), fill, jnp.int32)])
    e0 = flat[:NS * CA * CHUNK].reshape(NS, CA, CHUNK)
    e1 = flat[NS * CA * CHUNK:].reshape(NS, CB, CHUNK)
    if CB < CMAX:
        e1 = jnp.concatenate(
            [e1, jnp.zeros((NS, CMAX - CB, CHUNK), jnp.int32)], axis=1)
    if CA < CMAX:
        e0 = jnp.concatenate(
            [e0, jnp.zeros((NS, CMAX - CA, CHUNK), jnp.int32)], axis=1)
    return jnp.stack([e0, e1], axis=1).reshape(NW * CMAX, CHUNK)


def kernel(x, edge_index, W0, b0, W1, b1):
    xs0 = x[:, :DC]
    xs1 = x[:, DC:]
    srcp = _partition(edge_index[0], 0)
    dstp = _partition(edge_index[1], N)
    aggp0, degp = _sc_agg_deg(xs1, srcp, dstp)
    y0 = _tc_combine(aggp0, degp, xs0, W0, b0.reshape(1, DC))
    aggp1 = _sc_agg(y0, srcp, dstp)
    y1 = _tc_combine(aggp1, degp, xs1, W1, b1.reshape(1, DC))
    return jnp.concatenate([y0, y1], axis=-1)
